# baseline (speedup 1.0000x reference)
import functools

import jax
import jax.numpy as jnp
from jax import lax
from jax.experimental import pallas as pl
from jax.experimental.pallas import tpu as pltpu
from jax.experimental.pallas import tpu_sc as plsc

_N_FEATURES = 8
_FREQUENCY = 0.01
_GAIN = 1.0

_LANES = 16
_NWORKERS = 32
_CHUNK = 64


def _season_params(num_cols):
    key = jax.random.key(42)
    key, kf = jax.random.split(key)
    features = jax.random.randint(kf, (_N_FEATURES,), 0, num_cols)
    freqs, gains = [], []
    for _ in range(_N_FEATURES):
        key, k1, k2 = jax.random.split(key, 3)
        freqs.append(jax.random.uniform(k1, (), dtype=jnp.float32) * _FREQUENCY)
        gains.append(jax.random.uniform(k2, (), dtype=jnp.float32) * _GAIN)
    return features, jnp.stack(freqs), jnp.stack(gains)


def _sc_body(idx_hbm, phs_hbm, gain_hbm, hrow_hbm, in_hbm, out_hbm,
             idxv_ref, phsv_ref, gainv_ref, hrowv_ref, buf, *,
             rows_total, cols):
    wid = lax.axis_index("s") * 2 + lax.axis_index("c")
    stripe = rows_total // _NWORKERS
    base = wid * stripe

    pltpu.sync_copy(idx_hbm, idxv_ref)
    pltpu.sync_copy(phs_hbm, phsv_ref)
    pltpu.sync_copy(gain_hbm, gainv_ref)
    pltpu.sync_copy(hrow_hbm, hrowv_ref)

    @pl.loop(0, stripe // _CHUNK)
    def chunk_body(k):
        row0 = base + k * _CHUNK
        pltpu.sync_copy(in_hbm.at[pl.ds(row0 * cols, _CHUNK * cols)], buf)

        @pl.loop(0, _CHUNK // 2)
        def row_body(j):
            jrow = 2 * j
            idx = jnp.full((_LANES,), jrow * cols, jnp.int32) + idxv_ref[...]
            rf = (row0 + jrow).astype(jnp.float32)
            rglob = jnp.full((_LANES,), rf, jnp.float32) + hrowv_ref[...]
            x = rglob * phsv_ref[...]
            x2 = x * x
            p = x2 * jnp.float32(1.0 / 120.0) - jnp.float32(1.0 / 6.0)
            s = gainv_ref[...] * x * (x2 * p + jnp.float32(1.0))
            plsc.addupdate_scatter(buf, [idx], s)

        pltpu.sync_copy(buf, out_hbm.at[pl.ds(row0 * cols, _CHUNK * cols)])


def kernel(inp):
    rows, cols = inp.shape
    features, freqs, gains = _season_params(cols)
    feat2 = jnp.concatenate([features, features]).astype(jnp.int32)
    half = jnp.concatenate(
        [jnp.zeros((_N_FEATURES,)), jnp.ones((_N_FEATURES,))]
    )
    idx_base = feat2 + half.astype(jnp.int32) * jnp.int32(cols)
    phase2 = (
        jnp.concatenate([freqs, freqs])
        * jnp.float32(2.0 * 3.14)
        * jnp.float32(1.0 / rows)
    )
    gain2 = jnp.concatenate([gains, gains])
    half_f = half.astype(jnp.float32)

    mesh = plsc.VectorSubcoreMesh(core_axis_name="c", subcore_axis_name="s")
    body = functools.partial(_sc_body, rows_total=rows, cols=cols)
    out_flat = pl.kernel(
        body,
        out_type=jax.ShapeDtypeStruct((rows * cols,), jnp.float32),
        mesh=mesh,
        compiler_params=pltpu.CompilerParams(needs_layout_passes=False),
        scratch_types=[
            pltpu.VMEM((_LANES,), jnp.int32),
            pltpu.VMEM((_LANES,), jnp.float32),
            pltpu.VMEM((_LANES,), jnp.float32),
            pltpu.VMEM((_LANES,), jnp.float32),
            pltpu.VMEM((_CHUNK * cols,), jnp.float32),
        ],
    )(idx_base, phase2, gain2, half_f, inp.reshape(-1))
    return out_flat.reshape(rows, cols)


# speedup vs baseline: 1.0756x; 1.0756x over previous
import functools

import jax
import jax.numpy as jnp
from jax import lax
from jax.experimental import pallas as pl
from jax.experimental.pallas import tpu as pltpu
from jax.experimental.pallas import tpu_sc as plsc

_N_FEATURES = 8
_FREQUENCY = 0.01
_GAIN = 1.0

_LANES = 16
_NWORKERS = 32
_CHUNK = 32
_NBUF = 4


def _season_params(num_cols):
    key = jax.random.key(42)
    key, kf = jax.random.split(key)
    features = jax.random.randint(kf, (_N_FEATURES,), 0, num_cols)
    freqs, gains = [], []
    for _ in range(_N_FEATURES):
        key, k1, k2 = jax.random.split(key, 3)
        freqs.append(jax.random.uniform(k1, (), dtype=jnp.float32) * _FREQUENCY)
        gains.append(jax.random.uniform(k2, (), dtype=jnp.float32) * _GAIN)
    return features, jnp.stack(freqs), jnp.stack(gains)


def _sc_body(idx_hbm, phs_hbm, gain_hbm, hrow_hbm, in_hbm, out_hbm,
             idxv_ref, phsv_ref, gainv_ref, hrowv_ref, buf0, buf1, buf2,
             buf3, insem, outsem, *, rows_total, cols):
    wid = lax.axis_index("s") * 2 + lax.axis_index("c")
    stripe = rows_total // _NWORKERS
    base = wid * stripe

    pltpu.sync_copy(idx_hbm, idxv_ref)
    pltpu.sync_copy(phs_hbm, phsv_ref)
    pltpu.sync_copy(gain_hbm, gainv_ref)
    pltpu.sync_copy(hrow_hbm, hrowv_ref)

    nch = stripe // _CHUNK
    bufs = [buf0, buf1, buf2, buf3]

    def in_copy(c, b):
        return pltpu.make_async_copy(
            in_hbm.at[pl.ds((base + c * _CHUNK) * cols, _CHUNK * cols)],
            bufs[b], insem.at[b])

    def out_copy(c, b):
        return pltpu.make_async_copy(
            bufs[b],
            out_hbm.at[pl.ds((base + c * _CHUNK) * cols, _CHUNK * cols)],
            outsem.at[b])

    in_copy(0, 0).start()
    in_copy(1, 1).start()

    @pl.loop(0, nch // _NBUF)
    def superstep(s):
        for u in range(_NBUF):
            k = s * _NBUF + u
            in_copy(k, u).wait()
            row0 = base + k * _CHUNK

            @pl.loop(0, _CHUNK // 2)
            def row_body(j):
                jrow = 2 * j
                idx = (jnp.full((_LANES,), jrow * cols, jnp.int32)
                       + idxv_ref[...])
                rf = (row0 + jrow).astype(jnp.float32)
                rglob = jnp.full((_LANES,), rf, jnp.float32) + hrowv_ref[...]
                x = rglob * phsv_ref[...]
                x2 = x * x
                p = x2 * jnp.float32(1.0 / 120.0) - jnp.float32(1.0 / 6.0)
                s_v = gainv_ref[...] * x * (x2 * p + jnp.float32(1.0))
                plsc.addupdate_scatter(bufs[u], [idx], s_v)

            out_copy(k, u).start()

            @pl.when(k >= 2)
            def _():
                out_copy(k - 2, (u - 2) % _NBUF).wait()

            @pl.when(k + 2 < nch)
            def _():
                in_copy(k + 2, (u + 2) % _NBUF).start()

    out_copy(0, (_NBUF - 2) % _NBUF).wait()
    out_copy(0, (_NBUF - 1) % _NBUF).wait()


def kernel(inp):
    rows, cols = inp.shape
    features, freqs, gains = _season_params(cols)
    feat2 = jnp.concatenate([features, features]).astype(jnp.int32)
    half = jnp.concatenate(
        [jnp.zeros((_N_FEATURES,)), jnp.ones((_N_FEATURES,))]
    )
    idx_base = feat2 + half.astype(jnp.int32) * jnp.int32(cols)
    phase2 = (
        jnp.concatenate([freqs, freqs])
        * jnp.float32(2.0 * 3.14)
        * jnp.float32(1.0 / rows)
    )
    gain2 = jnp.concatenate([gains, gains])
    half_f = half.astype(jnp.float32)

    mesh = plsc.VectorSubcoreMesh(core_axis_name="c", subcore_axis_name="s")
    body = functools.partial(_sc_body, rows_total=rows, cols=cols)
    out_flat = pl.kernel(
        body,
        out_type=jax.ShapeDtypeStruct((rows * cols,), jnp.float32),
        mesh=mesh,
        compiler_params=pltpu.CompilerParams(needs_layout_passes=False),
        scratch_types=[
            pltpu.VMEM((_LANES,), jnp.int32),
            pltpu.VMEM((_LANES,), jnp.float32),
            pltpu.VMEM((_LANES,), jnp.float32),
            pltpu.VMEM((_LANES,), jnp.float32),
            pltpu.VMEM((_CHUNK * cols,), jnp.float32),
            pltpu.VMEM((_CHUNK * cols,), jnp.float32),
            pltpu.VMEM((_CHUNK * cols,), jnp.float32),
            pltpu.VMEM((_CHUNK * cols,), jnp.float32),
            pltpu.SemaphoreType.DMA((_NBUF,)),
            pltpu.SemaphoreType.DMA((_NBUF,)),
        ],
    )(idx_base, phase2, gain2, half_f, inp.reshape(-1))
    return out_flat.reshape(rows, cols)


# speedup vs baseline: 1.0794x; 1.0036x over previous
import functools

import jax
import jax.numpy as jnp
from jax import lax
from jax.experimental import pallas as pl
from jax.experimental.pallas import tpu as pltpu
from jax.experimental.pallas import tpu_sc as plsc

_N_FEATURES = 8
_FREQUENCY = 0.01
_GAIN = 1.0

_LANES = 16
_NWORKERS = 32
_CHUNK = 32
_NBUF = 4


def _season_params(num_cols):
    key = jax.random.key(42)
    key, kf = jax.random.split(key)
    features = jax.random.randint(kf, (_N_FEATURES,), 0, num_cols)
    freqs, gains = [], []
    for _ in range(_N_FEATURES):
        key, k1, k2 = jax.random.split(key, 3)
        freqs.append(jax.random.uniform(k1, (), dtype=jnp.float32) * _FREQUENCY)
        gains.append(jax.random.uniform(k2, (), dtype=jnp.float32) * _GAIN)
    return features, jnp.stack(freqs), jnp.stack(gains)


def _sc_body(idx_hbm, phs_hbm, gain_hbm, hrow_hbm, in_hbm, out_hbm,
             idxv_ref, phsv_ref, gainv_ref, hrowv_ref, buf0, buf1, buf2,
             buf3, insem, outsem, *, rows_total, cols):
    wid = lax.axis_index("s") * 2 + lax.axis_index("c")
    stripe = rows_total // _NWORKERS
    base = wid * stripe

    pltpu.sync_copy(idx_hbm, idxv_ref)
    pltpu.sync_copy(phs_hbm, phsv_ref)
    pltpu.sync_copy(gain_hbm, gainv_ref)
    pltpu.sync_copy(hrow_hbm, hrowv_ref)

    nch = stripe // _CHUNK
    bufs = [buf0, buf1, buf2, buf3]

    def in_copy(c, b):
        return pltpu.make_async_copy(
            in_hbm.at[pl.ds((base + c * _CHUNK) * cols, _CHUNK * cols)],
            bufs[b], insem.at[b])

    def out_copy(c, b):
        return pltpu.make_async_copy(
            bufs[b],
            out_hbm.at[pl.ds((base + c * _CHUNK) * cols, _CHUNK * cols)],
            outsem.at[b])

    in_copy(0, 0).start()
    in_copy(1, 1).start()

    @pl.loop(0, nch // _NBUF)
    def superstep(s):
        for u in range(_NBUF):
            k = s * _NBUF + u
            in_copy(k, u).wait()
            row0 = base + k * _CHUNK
            idxv = idxv_ref[...]
            phsv = phsv_ref[...]
            gainv = gainv_ref[...]
            rf = row0.astype(jnp.float32)
            rbase = jnp.full((_LANES,), rf, jnp.float32) + hrowv_ref[...]
            for jj in range(_CHUNK // 2):
                jrow = 2 * jj
                rglob = rbase + jnp.float32(jrow)
                x = rglob * phsv
                x2 = x * x
                p = x2 * jnp.float32(1.0 / 120.0) - jnp.float32(1.0 / 6.0)
                s_v = gainv * x * (x2 * p + jnp.float32(1.0))
                idx = idxv + jnp.int32(jrow * cols)
                plsc.addupdate_scatter(bufs[u], [idx], s_v)

            out_copy(k, u).start()

            @pl.when(k >= 2)
            def _():
                out_copy(k - 2, (u - 2) % _NBUF).wait()

            @pl.when(k + 2 < nch)
            def _():
                in_copy(k + 2, (u + 2) % _NBUF).start()

    out_copy(0, (_NBUF - 2) % _NBUF).wait()
    out_copy(0, (_NBUF - 1) % _NBUF).wait()


def kernel(inp):
    rows, cols = inp.shape
    features, freqs, gains = _season_params(cols)
    feat2 = jnp.concatenate([features, features]).astype(jnp.int32)
    half = jnp.concatenate(
        [jnp.zeros((_N_FEATURES,)), jnp.ones((_N_FEATURES,))]
    )
    idx_base = feat2 + half.astype(jnp.int32) * jnp.int32(cols)
    phase2 = (
        jnp.concatenate([freqs, freqs])
        * jnp.float32(2.0 * 3.14)
        * jnp.float32(1.0 / rows)
    )
    gain2 = jnp.concatenate([gains, gains])
    half_f = half.astype(jnp.float32)

    mesh = plsc.VectorSubcoreMesh(core_axis_name="c", subcore_axis_name="s")
    body = functools.partial(_sc_body, rows_total=rows, cols=cols)
    out_flat = pl.kernel(
        body,
        out_type=jax.ShapeDtypeStruct((rows * cols,), jnp.float32),
        mesh=mesh,
        compiler_params=pltpu.CompilerParams(needs_layout_passes=False),
        scratch_types=[
            pltpu.VMEM((_LANES,), jnp.int32),
            pltpu.VMEM((_LANES,), jnp.float32),
            pltpu.VMEM((_LANES,), jnp.float32),
            pltpu.VMEM((_LANES,), jnp.float32),
            pltpu.VMEM((_CHUNK * cols,), jnp.float32),
            pltpu.VMEM((_CHUNK * cols,), jnp.float32),
            pltpu.VMEM((_CHUNK * cols,), jnp.float32),
            pltpu.VMEM((_CHUNK * cols,), jnp.float32),
            pltpu.SemaphoreType.DMA((_NBUF,)),
            pltpu.SemaphoreType.DMA((_NBUF,)),
        ],
    )(idx_base, phase2, gain2, half_f, inp.reshape(-1))
    return out_flat.reshape(rows, cols)


# speedup vs baseline: 1.6582x; 1.5362x over previous
import functools

import jax
import jax.numpy as jnp
from jax import lax
from jax.experimental import pallas as pl
from jax.experimental.pallas import tpu as pltpu
from jax.experimental.pallas import tpu_sc as plsc

_N_FEATURES = 8
_FREQUENCY = 0.01
_GAIN = 1.0

_LANES = 16
_NWORKERS = 32
_CHUNK = 32
_NBUF = 4


def _season_params(num_cols):
    key = jax.random.key(42)
    key, kf = jax.random.split(key)
    features = jax.random.randint(kf, (_N_FEATURES,), 0, num_cols)
    freqs, gains = [], []
    for _ in range(_N_FEATURES):
        key, k1, k2 = jax.random.split(key, 3)
        freqs.append(jax.random.uniform(k1, (), dtype=jnp.float32) * _FREQUENCY)
        gains.append(jax.random.uniform(k2, (), dtype=jnp.float32) * _GAIN)
    return features, jnp.stack(freqs), jnp.stack(gains)


def _sc_body(idx_hbm, halfi_hbm, phs_hbm, gain_hbm, hrow_hbm, in_hbm,
             out_hbm, idxv_ref, halfi_ref, phsv_ref, gainv_ref, hrowv_ref,
             buf0, buf1, buf2, buf3, insem, outsem, *, rows_total, cols):
    wid = lax.axis_index("s") * 2 + lax.axis_index("c")
    stripe = rows_total // _NWORKERS
    base = wid * stripe

    pltpu.sync_copy(idx_hbm, idxv_ref)
    pltpu.sync_copy(halfi_hbm, halfi_ref)
    pltpu.sync_copy(phs_hbm, phsv_ref)
    pltpu.sync_copy(gain_hbm, gainv_ref)
    pltpu.sync_copy(hrow_hbm, hrowv_ref)

    nch = stripe // _CHUNK
    bufs = [buf0, buf1, buf2, buf3]

    def in_copy(c, b):
        return pltpu.make_async_copy(
            in_hbm.at[pl.ds(base + c * _CHUNK, _CHUNK), :],
            bufs[b], insem.at[b])

    def out_copy(c, b):
        return pltpu.make_async_copy(
            bufs[b],
            out_hbm.at[pl.ds(base + c * _CHUNK, _CHUNK), :],
            outsem.at[b])

    in_copy(0, 0).start()
    in_copy(1, 1).start()

    @pl.loop(0, nch // _NBUF)
    def superstep(s):
        for u in range(_NBUF):
            k = s * _NBUF + u
            in_copy(k, u).wait()
            row0 = base + k * _CHUNK
            featv = idxv_ref[...]
            halfi = halfi_ref[...]
            phsv = phsv_ref[...]
            gainv = gainv_ref[...]
            rf = row0.astype(jnp.float32)
            rbase = jnp.full((_LANES,), rf, jnp.float32) + hrowv_ref[...]
            for jj in range(_CHUNK // 2):
                jrow = 2 * jj
                rglob = rbase + jnp.float32(jrow)
                x = rglob * phsv
                x2 = x * x
                p = x2 * jnp.float32(1.0 / 120.0) - jnp.float32(1.0 / 6.0)
                s_v = gainv * x * (x2 * p + jnp.float32(1.0))
                rloc = halfi + jnp.int32(jrow)
                plsc.addupdate_scatter(bufs[u], [rloc, featv], s_v)

            out_copy(k, u).start()

            @pl.when(k >= 2)
            def _():
                out_copy(k - 2, (u - 2) % _NBUF).wait()

            @pl.when(k + 2 < nch)
            def _():
                in_copy(k + 2, (u + 2) % _NBUF).start()

    out_copy(0, (_NBUF - 2) % _NBUF).wait()
    out_copy(0, (_NBUF - 1) % _NBUF).wait()


def kernel(inp):
    rows, cols = inp.shape
    features, freqs, gains = _season_params(cols)
    feat2 = jnp.concatenate([features, features]).astype(jnp.int32)
    half = jnp.concatenate(
        [jnp.zeros((_N_FEATURES,)), jnp.ones((_N_FEATURES,))]
    )
    half_i = half.astype(jnp.int32)
    phase2 = (
        jnp.concatenate([freqs, freqs])
        * jnp.float32(2.0 * 3.14)
        * jnp.float32(1.0 / rows)
    )
    gain2 = jnp.concatenate([gains, gains])
    half_f = half.astype(jnp.float32)

    mesh = plsc.VectorSubcoreMesh(core_axis_name="c", subcore_axis_name="s")
    body = functools.partial(_sc_body, rows_total=rows, cols=cols)
    out_flat = pl.kernel(
        body,
        out_type=jax.ShapeDtypeStruct((rows, cols), jnp.float32),
        mesh=mesh,
        compiler_params=pltpu.CompilerParams(needs_layout_passes=False),
        scratch_types=[
            pltpu.VMEM((_LANES,), jnp.int32),
            pltpu.VMEM((_LANES,), jnp.int32),
            pltpu.VMEM((_LANES,), jnp.float32),
            pltpu.VMEM((_LANES,), jnp.float32),
            pltpu.VMEM((_LANES,), jnp.float32),
            pltpu.VMEM((_CHUNK, cols), jnp.float32),
            pltpu.VMEM((_CHUNK, cols), jnp.float32),
            pltpu.VMEM((_CHUNK, cols), jnp.float32),
            pltpu.VMEM((_CHUNK, cols), jnp.float32),
            pltpu.SemaphoreType.DMA((_NBUF,)),
            pltpu.SemaphoreType.DMA((_NBUF,)),
        ],
    )(feat2, half_i, phase2, gain2, half_f, inp)
    return out_flat


# speedup vs baseline: 1.6593x; 1.0007x over previous
import functools

import jax
import jax.numpy as jnp
from jax import lax
from jax.experimental import pallas as pl
from jax.experimental.pallas import tpu as pltpu
from jax.experimental.pallas import tpu_sc as plsc

_N_FEATURES = 8
_FREQUENCY = 0.01
_GAIN = 1.0

_LANES = 16
_NWORKERS = 32
_CHUNK = 32
_NBUF = 4


def _season_params(num_cols):
    key = jax.random.key(42)
    key, kf = jax.random.split(key)
    features = jax.random.randint(kf, (_N_FEATURES,), 0, num_cols)
    freqs, gains = [], []
    for _ in range(_N_FEATURES):
        key, k1, k2 = jax.random.split(key, 3)
        freqs.append(jax.random.uniform(k1, (), dtype=jnp.float32) * _FREQUENCY)
        gains.append(jax.random.uniform(k2, (), dtype=jnp.float32) * _GAIN)
    return features, jnp.stack(freqs), jnp.stack(gains)


def _sc_body(feat_hbm, halfi_hbm, phs_hbm, gain_hbm, hrow_hbm, in_hbm,
             out_hbm, featv_ref, halfi_ref, phsv_ref, gainv_ref, hrowv_ref,
             buf0, buf1, buf2, buf3, insem, outsem, *, rows_total, cols):
    wid = lax.axis_index("s") * 2 + lax.axis_index("c")
    stripe = rows_total // _NWORKERS
    base = wid * stripe

    pltpu.sync_copy(feat_hbm, featv_ref)
    pltpu.sync_copy(halfi_hbm, halfi_ref)
    pltpu.sync_copy(phs_hbm, phsv_ref)
    pltpu.sync_copy(gain_hbm, gainv_ref)
    pltpu.sync_copy(hrow_hbm, hrowv_ref)

    nch = stripe // _CHUNK
    bufs = [buf0, buf1, buf2, buf3]

    def in_copy(c, b):
        return pltpu.make_async_copy(
            in_hbm.at[pl.ds(base + c * _CHUNK, _CHUNK), :],
            bufs[b], insem.at[b])

    def out_copy(c, b):
        return pltpu.make_async_copy(
            bufs[b],
            out_hbm.at[pl.ds(base + c * _CHUNK, _CHUNK), :],
            outsem.at[b])

    in_copy(0, 0).start()
    in_copy(1, 1).start()

    @pl.loop(0, nch // _NBUF)
    def superstep(s):
        for u in range(_NBUF):
            k = s * _NBUF + u
            in_copy(k, u).wait()
            row0 = base + k * _CHUNK
            featv = featv_ref[...]
            halfi = halfi_ref[...]
            phsv = phsv_ref[...]
            gainv = gainv_ref[...]
            rf = row0.astype(jnp.float32)
            rbase = jnp.full((_LANES,), rf, jnp.float32) + hrowv_ref[...]
            for jj in range(_CHUNK // 2):
                jrow = 2 * jj
                rglob = rbase + jnp.float32(jrow)
                x = rglob * phsv
                x2 = x * x
                p = x2 * jnp.float32(1.0 / 120.0) - jnp.float32(1.0 / 6.0)
                s_v = gainv * x * (x2 * p + jnp.float32(1.0))
                rloc = halfi + jnp.int32(jrow)
                plsc.addupdate_scatter(bufs[u], [rloc, featv], s_v)

            out_copy(k, u).start()

            @pl.when(k >= 2)
            def _():
                out_copy(k - 2, (u - 2) % _NBUF).wait()

            @pl.when(k + 2 < nch)
            def _():
                in_copy(k + 2, (u + 2) % _NBUF).start()

    out_copy(0, (_NBUF - 2) % _NBUF).wait()
    out_copy(0, (_NBUF - 1) % _NBUF).wait()


def kernel(inp):
    rows, cols = inp.shape
    features, freqs, gains = _season_params(cols)
    feat2 = jnp.concatenate([features, features]).astype(jnp.int32)
    half = jnp.concatenate(
        [jnp.zeros((_N_FEATURES,)), jnp.ones((_N_FEATURES,))]
    )
    half_i = half.astype(jnp.int32)
    phase2 = (
        jnp.concatenate([freqs, freqs])
        * jnp.float32(2.0 * 3.14)
        * jnp.float32(1.0 / rows)
    )
    gain2 = jnp.concatenate([gains, gains])
    half_f = half.astype(jnp.float32)

    mesh = plsc.VectorSubcoreMesh(core_axis_name="c", subcore_axis_name="s")
    body = functools.partial(_sc_body, rows_total=rows, cols=cols)
    out = pl.kernel(
        body,
        out_type=jax.ShapeDtypeStruct((rows, cols), jnp.float32),
        mesh=mesh,
        compiler_params=pltpu.CompilerParams(needs_layout_passes=False),
        scratch_types=[
            pltpu.VMEM((_LANES,), jnp.int32),
            pltpu.VMEM((_LANES,), jnp.int32),
            pltpu.VMEM((_LANES,), jnp.float32),
            pltpu.VMEM((_LANES,), jnp.float32),
            pltpu.VMEM((_LANES,), jnp.float32),
            pltpu.VMEM((_CHUNK, cols), jnp.float32),
            pltpu.VMEM((_CHUNK, cols), jnp.float32),
            pltpu.VMEM((_CHUNK, cols), jnp.float32),
            pltpu.VMEM((_CHUNK, cols), jnp.float32),
            pltpu.SemaphoreType.DMA((_NBUF,)),
            pltpu.SemaphoreType.DMA((_NBUF,)),
        ],
    )(feat2, half_i, phase2, gain2, half_f, inp)
    return out


# speedup vs baseline: 1.6610x; 1.0010x over previous
import functools

import jax
import jax.numpy as jnp
from jax import lax
from jax.experimental import pallas as pl
from jax.experimental.pallas import tpu as pltpu
from jax.experimental.pallas import tpu_sc as plsc

_N_FEATURES = 8
_FREQUENCY = 0.01
_GAIN = 1.0

_LANES = 16
_NWORKERS = 32
_CHUNK = 16
_NBUF = 8
_LEAD = 4


def _season_params(num_cols):
    key = jax.random.key(42)
    key, kf = jax.random.split(key)
    features = jax.random.randint(kf, (_N_FEATURES,), 0, num_cols)
    freqs, gains = [], []
    for _ in range(_N_FEATURES):
        key, k1, k2 = jax.random.split(key, 3)
        freqs.append(jax.random.uniform(k1, (), dtype=jnp.float32) * _FREQUENCY)
        gains.append(jax.random.uniform(k2, (), dtype=jnp.float32) * _GAIN)
    return features, jnp.stack(freqs), jnp.stack(gains)


def _sc_body(feat_hbm, halfi_hbm, phs_hbm, gain_hbm, hrow_hbm, in_hbm,
             out_hbm, featv_ref, halfi_ref, phsv_ref, gainv_ref, hrowv_ref,
             buf0, buf1, buf2, buf3, buf4, buf5, buf6, buf7, insem,
             outsem, *, rows_total, cols):
    wid = lax.axis_index("s") * 2 + lax.axis_index("c")
    stripe = rows_total // _NWORKERS
    base = wid * stripe

    pltpu.sync_copy(feat_hbm, featv_ref)
    pltpu.sync_copy(halfi_hbm, halfi_ref)
    pltpu.sync_copy(phs_hbm, phsv_ref)
    pltpu.sync_copy(gain_hbm, gainv_ref)
    pltpu.sync_copy(hrow_hbm, hrowv_ref)

    nch = stripe // _CHUNK
    bufs = [buf0, buf1, buf2, buf3, buf4, buf5, buf6, buf7]

    def in_copy(c, b):
        return pltpu.make_async_copy(
            in_hbm.at[pl.ds(base + c * _CHUNK, _CHUNK), :],
            bufs[b], insem.at[b])

    def out_copy(c, b):
        return pltpu.make_async_copy(
            bufs[b],
            out_hbm.at[pl.ds(base + c * _CHUNK, _CHUNK), :],
            outsem.at[b])

    for b in range(_LEAD):
        in_copy(b, b).start()

    @pl.loop(0, nch // _NBUF)
    def superstep(s):
        for u in range(_NBUF):
            k = s * _NBUF + u
            in_copy(k, u).wait()
            row0 = base + k * _CHUNK
            featv = featv_ref[...]
            halfi = halfi_ref[...]
            phsv = phsv_ref[...]
            gainv = gainv_ref[...]
            rf = row0.astype(jnp.float32)
            rbase = jnp.full((_LANES,), rf, jnp.float32) + hrowv_ref[...]
            for jj in range(_CHUNK // 2):
                jrow = 2 * jj
                rglob = rbase + jnp.float32(jrow)
                x = rglob * phsv
                x2 = x * x
                p = x2 * jnp.float32(1.0 / 120.0) - jnp.float32(1.0 / 6.0)
                s_v = gainv * x * (x2 * p + jnp.float32(1.0))
                rloc = halfi + jnp.int32(jrow)
                plsc.addupdate_scatter(bufs[u], [rloc, featv], s_v)

            out_copy(k, u).start()

            @pl.when(k >= _LEAD)
            def _():
                out_copy(k - _LEAD, (u - _LEAD) % _NBUF).wait()

            @pl.when(k + _LEAD < nch)
            def _():
                in_copy(k + _LEAD, (u + _LEAD) % _NBUF).start()

    for i in range(_LEAD):
        out_copy(0, (nch - _LEAD + i) % _NBUF).wait()


def kernel(inp):
    rows, cols = inp.shape
    features, freqs, gains = _season_params(cols)
    feat2 = jnp.concatenate([features, features]).astype(jnp.int32)
    half = jnp.concatenate(
        [jnp.zeros((_N_FEATURES,)), jnp.ones((_N_FEATURES,))]
    )
    half_i = half.astype(jnp.int32)
    phase2 = (
        jnp.concatenate([freqs, freqs])
        * jnp.float32(2.0 * 3.14)
        * jnp.float32(1.0 / rows)
    )
    gain2 = jnp.concatenate([gains, gains])
    half_f = half.astype(jnp.float32)

    mesh = plsc.VectorSubcoreMesh(core_axis_name="c", subcore_axis_name="s")
    body = functools.partial(_sc_body, rows_total=rows, cols=cols)
    out = pl.kernel(
        body,
        out_type=jax.ShapeDtypeStruct((rows, cols), jnp.float32),
        mesh=mesh,
        compiler_params=pltpu.CompilerParams(needs_layout_passes=False),
        scratch_types=[
            pltpu.VMEM((_LANES,), jnp.int32),
            pltpu.VMEM((_LANES,), jnp.int32),
            pltpu.VMEM((_LANES,), jnp.float32),
            pltpu.VMEM((_LANES,), jnp.float32),
            pltpu.VMEM((_LANES,), jnp.float32),
            pltpu.VMEM((_CHUNK, cols), jnp.float32),
            pltpu.VMEM((_CHUNK, cols), jnp.float32),
            pltpu.VMEM((_CHUNK, cols), jnp.float32),
            pltpu.VMEM((_CHUNK, cols), jnp.float32),
            pltpu.VMEM((_CHUNK, cols), jnp.float32),
            pltpu.VMEM((_CHUNK, cols), jnp.float32),
            pltpu.VMEM((_CHUNK, cols), jnp.float32),
            pltpu.VMEM((_CHUNK, cols), jnp.float32),
            pltpu.SemaphoreType.DMA((_NBUF,)),
            pltpu.SemaphoreType.DMA((_NBUF,)),
        ],
    )(feat2, half_i, phase2, gain2, half_f, inp)
    return out
